# baseline (speedup 1.0000x reference)
import functools

import jax
import jax.numpy as jnp
from jax import lax
from jax.experimental import pallas as pl
from jax.experimental.pallas import tpu as pltpu
from jax.experimental.pallas import tpu_sc as plsc

_BETA = 0.25
_BB = 128


def _vq_body(e_ref, esq_ref, w2t_ref, wsq_ref, idx_ref, loss_ref):
    e = e_ref[...]
    bb = e.shape[0]
    mm2 = jnp.dot(e, w2t_ref[...],
                  preferred_element_type=jnp.float32)
    k = mm2.shape[1]
    ns = k // 128
    esq_b = jnp.broadcast_to(esq_ref[...], (bb, 128))
    wsq = wsq_ref[...]
    lane = lax.broadcasted_iota(jnp.int32, (bb, 128), 1)

    def half_argmin(s_lo, s_hi):
        nacc = 4
        avs = [jnp.full((bb, 128), jnp.inf, jnp.float32) for _ in range(nacc)]
        ais = [jnp.zeros((bb, 128), jnp.int32) for _ in range(nacc)]
        for s in range(s_lo, s_hi):
            a = s % nacc
            d2s = (esq_b - mm2[:, s * 128:(s + 1) * 128]) + jnp.broadcast_to(
                wsq[:, s * 128:(s + 1) * 128], (bb, 128))
            upd = d2s < avs[a]
            avs[a] = jnp.where(upd, d2s, avs[a])
            ais[a] = jnp.where(upd, s, ais[a])
        av, ai = avs[0], ais[0]
        for a in range(1, nacc):
            take = (avs[a] < av) | ((avs[a] == av) & (ais[a] < ai))
            av = jnp.where(take, avs[a], av)
            ai = jnp.where(take, ais[a], ai)
        v = jnp.min(av, axis=1, keepdims=True)
        kfull = ai * 128 + lane
        i = jnp.min(jnp.where(av == v, kfull, k), axis=1)
        return v, i

    vl, il = half_argmin(0, ns // 2)
    vr, ir = half_argmin(ns // 2, ns)
    vl_bf = vl.astype(jnp.bfloat16).astype(jnp.float32)
    pickr = vr < vl_bf
    idx_ref[0, 0, :] = jnp.where(pickr[:, 0], ir, il)
    dsel = jnp.where(pickr[:, 0], vr[:, 0], vl[:, 0])
    @pl.when(pl.program_id(0) == 0)
    def _():
        loss_ref[...] = jnp.zeros((1, 1), jnp.float32)
    loss_ref[...] += jnp.sum(dsel).reshape(1, 1)


def _argmin_losses(encoding, embedding_weight):
    b, d = encoding.shape
    k = embedding_weight.shape[0]
    enc_sq = jnp.sum(encoding * encoding, axis=1, keepdims=True)
    w_sq = jnp.sum(embedding_weight * embedding_weight, axis=1)
    w2t = (2.0 * embedding_weight).T
    g = b // _BB
    idx3, loss = pl.pallas_call(
        _vq_body,
        grid=(g,),
        in_specs=[
            pl.BlockSpec((_BB, d), lambda i: (i, 0)),
            pl.BlockSpec((_BB, 1), lambda i: (i, 0)),
            pl.BlockSpec((d, k), lambda i: (0, 0)),
            pl.BlockSpec((1, k), lambda i: (0, 0)),
        ],
        out_specs=[
            pl.BlockSpec((1, 1, _BB), lambda i: (i, 0, 0)),
            pl.BlockSpec((1, 1), lambda i: (0, 0)),
        ],
        out_shape=[
            jax.ShapeDtypeStruct((g, 1, _BB), jnp.int32),
            jax.ShapeDtypeStruct((1, 1), jnp.float32),
        ],
        compiler_params=pltpu.CompilerParams(
            dimension_semantics=("arbitrary",),
        ),
    )(encoding, enc_sq, w2t, w_sq.reshape(1, k))
    return idx3.reshape(b), loss[0, 0]


def _make_sc_gather(b, k, d):
    info = plsc.get_sparse_core_info()
    nw = info.num_cores * info.num_subcores
    b_per_w = b // nw
    nc = b_per_w // 128
    nbuf = 6
    mesh = plsc.VectorSubcoreMesh(core_axis_name="c", subcore_axis_name="s")

    @functools.partial(
        pl.kernel, mesh=mesh,
        out_type=jax.ShapeDtypeStruct((b, 128), jnp.float32),
        scratch_types=(
            [pltpu.VMEM((b_per_w,), jnp.int32)]
            + [pltpu.VMEM((128, 128), jnp.float32)] * nbuf
            + [pltpu.SemaphoreType.DMA] * (2 * nbuf)
        ),
    )
    def gather_kernel(table_hbm, idx_hbm, out_hbm, idx_v, *rest):
        bufs = rest[:nbuf]
        gsem = rest[nbuf:2 * nbuf]
        wsem = rest[2 * nbuf:3 * nbuf]
        wid = lax.axis_index("s") * info.num_cores + lax.axis_index("c")
        base = wid * b_per_w
        pltpu.sync_copy(idx_hbm.at[pl.ds(base, b_per_w)], idx_v)

        hg = [None] * nbuf
        hw = [None] * nbuf
        for j in range(min(nbuf, nc)):
            hg[j] = pltpu.async_copy(
                table_hbm.at[idx_v.at[pl.ds(j * 128, 128)]], bufs[j], gsem[j])
        for j in range(nc):
            q = j % nbuf
            hg[q].wait()
            hw[q] = pltpu.async_copy(
                bufs[q], out_hbm.at[pl.ds(base + j * 128, 128)], wsem[q])
            nxt = j + nbuf
            if nxt < nc:
                hw[q].wait()
                hg[q] = pltpu.async_copy(
                    table_hbm.at[idx_v.at[pl.ds(nxt * 128, 128)]],
                    bufs[q], gsem[q])
        for j in range(max(0, nc - nbuf), nc):
            hw[j % nbuf].wait()

    return gather_kernel


def kernel(encoding, embedding_weight):
    b, d = encoding.shape
    k = embedding_weight.shape[0]
    quantized_index, loss_sum = _argmin_losses(encoding, embedding_weight)
    table_p = jnp.pad(embedding_weight, ((0, 0), (0, 128 - d)))
    quantized_st = _make_sc_gather(b, k, d)(table_p, quantized_index)[:, :d]
    s = loss_sum / (b * d)
    commitment_loss = s
    embedding_loss = s
    vq_loss = commitment_loss * _BETA + embedding_loss
    return (quantized_index, quantized_st, vq_loss, embedding_loss, commitment_loss)


# speedup vs baseline: 1.2248x; 1.2248x over previous
import functools

import jax
import jax.numpy as jnp
from jax import lax
from jax.experimental import pallas as pl
from jax.experimental.pallas import tpu as pltpu
from jax.experimental.pallas import tpu_sc as plsc

_BETA = 0.25
_BB = 128


def _vq_body(e_ref, esq_ref, w2t_ref, wsq_ref, idx_ref, loss_ref):
    e = e_ref[...]
    bb = e.shape[0]
    mm2 = jnp.dot(e, w2t_ref[...],
                  preferred_element_type=jnp.float32)
    k = mm2.shape[1]
    ns = k // 128
    esq_b = jnp.broadcast_to(esq_ref[...], (bb, 128))
    wsq = wsq_ref[...]
    lane = lax.broadcasted_iota(jnp.int32, (bb, 128), 1)

    def half_argmin(s_lo, s_hi):
        nacc = 1
        avs = [jnp.full((bb, 128), jnp.inf, jnp.float32) for _ in range(nacc)]
        ais = [jnp.zeros((bb, 128), jnp.int32) for _ in range(nacc)]
        for s in range(s_lo, s_hi):
            a = s % nacc
            d2s = (esq_b - mm2[:, s * 128:(s + 1) * 128]) + jnp.broadcast_to(
                wsq[:, s * 128:(s + 1) * 128], (bb, 128))
            upd = d2s < avs[a]
            avs[a] = jnp.where(upd, d2s, avs[a])
            ais[a] = jnp.where(upd, s, ais[a])
        av, ai = avs[0], ais[0]
        for a in range(1, nacc):
            take = (avs[a] < av) | ((avs[a] == av) & (ais[a] < ai))
            av = jnp.where(take, avs[a], av)
            ai = jnp.where(take, ais[a], ai)
        v = jnp.min(av, axis=1, keepdims=True)
        kfull = ai * 128 + lane
        i = jnp.min(jnp.where(av == v, kfull, k), axis=1)
        return v, i

    vl, il = half_argmin(0, ns // 2)
    vr, ir = half_argmin(ns // 2, ns)
    vl_bf = vl.astype(jnp.bfloat16).astype(jnp.float32)
    pickr = vr < vl_bf
    idx_ref[0, 0, :] = jnp.where(pickr[:, 0], ir, il)
    dsel = jnp.where(pickr[:, 0], vr[:, 0], vl[:, 0])
    @pl.when(pl.program_id(0) == 0)
    def _():
        loss_ref[...] = jnp.zeros((1, 1), jnp.float32)
    loss_ref[...] += jnp.sum(dsel).reshape(1, 1)


def _argmin_losses(encoding, embedding_weight):
    b, d = encoding.shape
    k = embedding_weight.shape[0]
    enc_sq = jnp.sum(encoding * encoding, axis=1, keepdims=True)
    w_sq = jnp.sum(embedding_weight * embedding_weight, axis=1)
    w2t = (2.0 * embedding_weight).T
    g = b // _BB
    idx3, loss = pl.pallas_call(
        _vq_body,
        grid=(g,),
        in_specs=[
            pl.BlockSpec((_BB, d), lambda i: (i, 0)),
            pl.BlockSpec((_BB, 1), lambda i: (i, 0)),
            pl.BlockSpec((d, k), lambda i: (0, 0)),
            pl.BlockSpec((1, k), lambda i: (0, 0)),
        ],
        out_specs=[
            pl.BlockSpec((1, 1, _BB), lambda i: (i, 0, 0)),
            pl.BlockSpec((1, 1), lambda i: (0, 0)),
        ],
        out_shape=[
            jax.ShapeDtypeStruct((g, 1, _BB), jnp.int32),
            jax.ShapeDtypeStruct((1, 1), jnp.float32),
        ],
        compiler_params=pltpu.CompilerParams(
            dimension_semantics=("arbitrary",),
        ),
    )(encoding, enc_sq, w2t, w_sq.reshape(1, k))
    return idx3.reshape(b), loss[0, 0]


def _make_sc_gather(b, k, d):
    info = plsc.get_sparse_core_info()
    nw = info.num_cores * info.num_subcores
    b_per_w = b // nw
    nc = b_per_w // 128
    nbuf = 6
    mesh = plsc.VectorSubcoreMesh(core_axis_name="c", subcore_axis_name="s")

    @functools.partial(
        pl.kernel, mesh=mesh,
        out_type=jax.ShapeDtypeStruct((b, 128), jnp.float32),
        scratch_types=(
            [pltpu.VMEM((b_per_w,), jnp.int32)]
            + [pltpu.VMEM((128, 128), jnp.float32)] * nbuf
            + [pltpu.SemaphoreType.DMA] * (2 * nbuf)
        ),
    )
    def gather_kernel(table_hbm, idx_hbm, out_hbm, idx_v, *rest):
        bufs = rest[:nbuf]
        gsem = rest[nbuf:2 * nbuf]
        wsem = rest[2 * nbuf:3 * nbuf]
        wid = lax.axis_index("s") * info.num_cores + lax.axis_index("c")
        base = wid * b_per_w
        pltpu.sync_copy(idx_hbm.at[pl.ds(base, b_per_w)], idx_v)

        hg = [None] * nbuf
        hw = [None] * nbuf
        for j in range(min(nbuf, nc)):
            hg[j] = pltpu.async_copy(
                table_hbm.at[idx_v.at[pl.ds(j * 128, 128)]], bufs[j], gsem[j])
        for j in range(nc):
            q = j % nbuf
            hg[q].wait()
            hw[q] = pltpu.async_copy(
                bufs[q], out_hbm.at[pl.ds(base + j * 128, 128)], wsem[q])
            nxt = j + nbuf
            if nxt < nc:
                hw[q].wait()
                hg[q] = pltpu.async_copy(
                    table_hbm.at[idx_v.at[pl.ds(nxt * 128, 128)]],
                    bufs[q], gsem[q])
        for j in range(max(0, nc - nbuf), nc):
            hw[j % nbuf].wait()

    return gather_kernel


def kernel(encoding, embedding_weight):
    b, d = encoding.shape
    k = embedding_weight.shape[0]
    quantized_index, loss_sum = _argmin_losses(encoding, embedding_weight)
    table_p = jnp.pad(embedding_weight, ((0, 0), (0, 128 - d)))
    quantized_st = _make_sc_gather(b, k, d)(table_p, quantized_index)[:, :d]
    s = loss_sum / (b * d)
    commitment_loss = s
    embedding_loss = s
    vq_loss = commitment_loss * _BETA + embedding_loss
    return (quantized_index, quantized_st, vq_loss, embedding_loss, commitment_loss)


# speedup vs baseline: 1.3467x; 1.0995x over previous
import functools

import jax
import jax.numpy as jnp
from jax import lax
from jax.experimental import pallas as pl
from jax.experimental.pallas import tpu as pltpu
from jax.experimental.pallas import tpu_sc as plsc

_BETA = 0.25
_BB = 256


def _vq_body(e_ref, esq_ref, w2t_ref, wsq_ref, idx_ref, loss_ref):
    e = e_ref[...]
    bb = e.shape[0]
    mm2 = jnp.dot(e, w2t_ref[...],
                  preferred_element_type=jnp.float32)
    k = mm2.shape[1]
    ns = k // 128
    esq_b = jnp.broadcast_to(esq_ref[...], (bb, 128))
    wsq = wsq_ref[...]
    lane = lax.broadcasted_iota(jnp.int32, (bb, 128), 1)

    def half_argmin(s_lo, s_hi):
        nacc = 1
        avs = [jnp.full((bb, 128), jnp.inf, jnp.float32) for _ in range(nacc)]
        ais = [jnp.zeros((bb, 128), jnp.int32) for _ in range(nacc)]
        for s in range(s_lo, s_hi):
            a = s % nacc
            d2s = (esq_b - mm2[:, s * 128:(s + 1) * 128]) + jnp.broadcast_to(
                wsq[:, s * 128:(s + 1) * 128], (bb, 128))
            upd = d2s < avs[a]
            avs[a] = jnp.where(upd, d2s, avs[a])
            ais[a] = jnp.where(upd, s, ais[a])
        av, ai = avs[0], ais[0]
        for a in range(1, nacc):
            take = (avs[a] < av) | ((avs[a] == av) & (ais[a] < ai))
            av = jnp.where(take, avs[a], av)
            ai = jnp.where(take, ais[a], ai)
        v = jnp.min(av, axis=1, keepdims=True)
        kfull = ai * 128 + lane
        i = jnp.min(jnp.where(av == v, kfull, k), axis=1)
        return v, i

    vl, il = half_argmin(0, ns // 2)
    vr, ir = half_argmin(ns // 2, ns)
    vl_bf = vl.astype(jnp.bfloat16).astype(jnp.float32)
    pickr = vr < vl_bf
    idx_ref[0, 0, :] = jnp.where(pickr[:, 0], ir, il)
    dsel = jnp.where(pickr[:, 0], vr[:, 0], vl[:, 0])
    @pl.when(pl.program_id(0) == 0)
    def _():
        loss_ref[...] = jnp.zeros((1, 1), jnp.float32)
    loss_ref[...] += jnp.sum(dsel).reshape(1, 1)


def _argmin_losses(encoding, embedding_weight):
    b, d = encoding.shape
    k = embedding_weight.shape[0]
    enc_sq = jnp.sum(encoding * encoding, axis=1, keepdims=True)
    w_sq = jnp.sum(embedding_weight * embedding_weight, axis=1)
    w2t = (2.0 * embedding_weight).T
    g = b // _BB
    idx3, loss = pl.pallas_call(
        _vq_body,
        grid=(g,),
        in_specs=[
            pl.BlockSpec((_BB, d), lambda i: (i, 0)),
            pl.BlockSpec((_BB, 1), lambda i: (i, 0)),
            pl.BlockSpec((d, k), lambda i: (0, 0)),
            pl.BlockSpec((1, k), lambda i: (0, 0)),
        ],
        out_specs=[
            pl.BlockSpec((1, 1, _BB), lambda i: (i, 0, 0)),
            pl.BlockSpec((1, 1), lambda i: (0, 0)),
        ],
        out_shape=[
            jax.ShapeDtypeStruct((g, 1, _BB), jnp.int32),
            jax.ShapeDtypeStruct((1, 1), jnp.float32),
        ],
        compiler_params=pltpu.CompilerParams(
            dimension_semantics=("arbitrary",),
        ),
    )(encoding, enc_sq, w2t, w_sq.reshape(1, k))
    return idx3.reshape(b), loss[0, 0]


def _make_sc_gather(b, k, d):
    info = plsc.get_sparse_core_info()
    nw = info.num_cores * info.num_subcores
    b_per_w = b // nw
    nc = b_per_w // 128
    nbuf = 6
    mesh = plsc.VectorSubcoreMesh(core_axis_name="c", subcore_axis_name="s")

    @functools.partial(
        pl.kernel, mesh=mesh,
        out_type=jax.ShapeDtypeStruct((b, 128), jnp.float32),
        scratch_types=(
            [pltpu.VMEM((b_per_w,), jnp.int32)]
            + [pltpu.VMEM((128, 128), jnp.float32)] * nbuf
            + [pltpu.SemaphoreType.DMA] * (2 * nbuf)
        ),
    )
    def gather_kernel(table_hbm, idx_hbm, out_hbm, idx_v, *rest):
        bufs = rest[:nbuf]
        gsem = rest[nbuf:2 * nbuf]
        wsem = rest[2 * nbuf:3 * nbuf]
        wid = lax.axis_index("s") * info.num_cores + lax.axis_index("c")
        base = wid * b_per_w
        pltpu.sync_copy(idx_hbm.at[pl.ds(base, b_per_w)], idx_v)

        hg = [None] * nbuf
        hw = [None] * nbuf
        for j in range(min(nbuf, nc)):
            hg[j] = pltpu.async_copy(
                table_hbm.at[idx_v.at[pl.ds(j * 128, 128)]], bufs[j], gsem[j])
        for j in range(nc):
            q = j % nbuf
            hg[q].wait()
            hw[q] = pltpu.async_copy(
                bufs[q], out_hbm.at[pl.ds(base + j * 128, 128)], wsem[q])
            nxt = j + nbuf
            if nxt < nc:
                hw[q].wait()
                hg[q] = pltpu.async_copy(
                    table_hbm.at[idx_v.at[pl.ds(nxt * 128, 128)]],
                    bufs[q], gsem[q])
        for j in range(max(0, nc - nbuf), nc):
            hw[j % nbuf].wait()

    return gather_kernel


def kernel(encoding, embedding_weight):
    b, d = encoding.shape
    k = embedding_weight.shape[0]
    quantized_index, loss_sum = _argmin_losses(encoding, embedding_weight)
    table_p = jnp.pad(embedding_weight, ((0, 0), (0, 128 - d)))
    quantized_st = _make_sc_gather(b, k, d)(table_p, quantized_index)[:, :d]
    s = loss_sum / (b * d)
    commitment_loss = s
    embedding_loss = s
    vq_loss = commitment_loss * _BETA + embedding_loss
    return (quantized_index, quantized_st, vq_loss, embedding_loss, commitment_loss)


# speedup vs baseline: 1.4119x; 1.0484x over previous
import functools

import jax
import jax.numpy as jnp
from jax import lax
from jax.experimental import pallas as pl
from jax.experimental.pallas import tpu as pltpu
from jax.experimental.pallas import tpu_sc as plsc

_BETA = 0.25
_BB = 256


def _vq_body(e_ref, esq_ref, w2t_ref, wsq_ref, idx_ref, loss_ref):
    e = e_ref[...]
    mm2_full = jnp.dot(e, w2t_ref[...],
                       preferred_element_type=jnp.float32)
    k = mm2_full.shape[1]
    ns = k // 128
    bb = 128
    wsq = wsq_ref[...]
    lane = lax.broadcasted_iota(jnp.int32, (bb, 128), 1)

    def half_argmin(mm2, esq_b, s_lo, s_hi):
        nacc = 1
        avs = [jnp.full((bb, 128), jnp.inf, jnp.float32) for _ in range(nacc)]
        ais = [jnp.zeros((bb, 128), jnp.int32) for _ in range(nacc)]
        for s in range(s_lo, s_hi):
            a = s % nacc
            d2s = (esq_b - mm2[:, s * 128:(s + 1) * 128]) + jnp.broadcast_to(
                wsq[:, s * 128:(s + 1) * 128], (bb, 128))
            upd = d2s < avs[a]
            avs[a] = jnp.where(upd, d2s, avs[a])
            ais[a] = jnp.where(upd, s, ais[a])
        av, ai = avs[0], ais[0]
        for a in range(1, nacc):
            take = (avs[a] < av) | ((avs[a] == av) & (ais[a] < ai))
            av = jnp.where(take, avs[a], av)
            ai = jnp.where(take, ais[a], ai)
        v = jnp.min(av, axis=1, keepdims=True)
        kfull = ai * 128 + lane
        i = jnp.min(jnp.where(av == v, kfull, k), axis=1)
        return v, i

    acc = jnp.zeros((1, 1), jnp.float32)
    for rb in range(_BB // bb):
        rows = slice(rb * bb, (rb + 1) * bb)
        mm2 = mm2_full[rows, :]
        esq_b = jnp.broadcast_to(esq_ref[rows, :], (bb, 128))
        vl, il = half_argmin(mm2, esq_b, 0, ns // 2)
        vr, ir = half_argmin(mm2, esq_b, ns // 2, ns)
        vl_bf = vl.astype(jnp.bfloat16).astype(jnp.float32)
        pickr = vr < vl_bf
        idx_ref[0, 0, rows] = jnp.where(pickr[:, 0], ir, il)
        dsel = jnp.where(pickr[:, 0], vr[:, 0], vl[:, 0])
        acc = acc + jnp.sum(dsel).reshape(1, 1)
    @pl.when(pl.program_id(0) == 0)
    def _():
        loss_ref[...] = jnp.zeros((1, 1), jnp.float32)
    loss_ref[...] += acc


def _argmin_losses(encoding, embedding_weight):
    b, d = encoding.shape
    k = embedding_weight.shape[0]
    enc_sq = jnp.sum(encoding * encoding, axis=1, keepdims=True)
    w_sq = jnp.sum(embedding_weight * embedding_weight, axis=1)
    w2t = (2.0 * embedding_weight).T
    g = b // _BB
    idx3, loss = pl.pallas_call(
        _vq_body,
        grid=(g,),
        in_specs=[
            pl.BlockSpec((_BB, d), lambda i: (i, 0)),
            pl.BlockSpec((_BB, 1), lambda i: (i, 0)),
            pl.BlockSpec((d, k), lambda i: (0, 0)),
            pl.BlockSpec((1, k), lambda i: (0, 0)),
        ],
        out_specs=[
            pl.BlockSpec((1, 1, _BB), lambda i: (i, 0, 0)),
            pl.BlockSpec((1, 1), lambda i: (0, 0)),
        ],
        out_shape=[
            jax.ShapeDtypeStruct((g, 1, _BB), jnp.int32),
            jax.ShapeDtypeStruct((1, 1), jnp.float32),
        ],
        compiler_params=pltpu.CompilerParams(
            dimension_semantics=("arbitrary",),
        ),
    )(encoding, enc_sq, w2t, w_sq.reshape(1, k))
    return idx3.reshape(b), loss[0, 0]


def _make_sc_gather(b, k, d):
    info = plsc.get_sparse_core_info()
    nw = info.num_cores * info.num_subcores
    b_per_w = b // nw
    nc = b_per_w // 128
    nbuf = 6
    mesh = plsc.VectorSubcoreMesh(core_axis_name="c", subcore_axis_name="s")

    @functools.partial(
        pl.kernel, mesh=mesh,
        out_type=jax.ShapeDtypeStruct((b, 128), jnp.float32),
        scratch_types=(
            [pltpu.VMEM((b_per_w,), jnp.int32)]
            + [pltpu.VMEM((128, 128), jnp.float32)] * nbuf
            + [pltpu.SemaphoreType.DMA] * (2 * nbuf)
        ),
    )
    def gather_kernel(table_hbm, idx_hbm, out_hbm, idx_v, *rest):
        bufs = rest[:nbuf]
        gsem = rest[nbuf:2 * nbuf]
        wsem = rest[2 * nbuf:3 * nbuf]
        wid = lax.axis_index("s") * info.num_cores + lax.axis_index("c")
        base = wid * b_per_w
        pltpu.sync_copy(idx_hbm.at[pl.ds(base, b_per_w)], idx_v)

        hg = [None] * nbuf
        hw = [None] * nbuf
        for j in range(min(nbuf, nc)):
            hg[j] = pltpu.async_copy(
                table_hbm.at[idx_v.at[pl.ds(j * 128, 128)]], bufs[j], gsem[j])
        for j in range(nc):
            q = j % nbuf
            hg[q].wait()
            hw[q] = pltpu.async_copy(
                bufs[q], out_hbm.at[pl.ds(base + j * 128, 128)], wsem[q])
            nxt = j + nbuf
            if nxt < nc:
                hw[q].wait()
                hg[q] = pltpu.async_copy(
                    table_hbm.at[idx_v.at[pl.ds(nxt * 128, 128)]],
                    bufs[q], gsem[q])
        for j in range(max(0, nc - nbuf), nc):
            hw[j % nbuf].wait()

    return gather_kernel


def kernel(encoding, embedding_weight):
    b, d = encoding.shape
    k = embedding_weight.shape[0]
    quantized_index, loss_sum = _argmin_losses(encoding, embedding_weight)
    table_p = jnp.pad(embedding_weight, ((0, 0), (0, 128 - d)))
    quantized_st = _make_sc_gather(b, k, d)(table_p, quantized_index)[:, :d]
    s = loss_sum / (b * d)
    commitment_loss = s
    embedding_loss = s
    vq_loss = commitment_loss * _BETA + embedding_loss
    return (quantized_index, quantized_st, vq_loss, embedding_loss, commitment_loss)


# speedup vs baseline: 1.4873x; 1.0534x over previous
import functools

import jax
import jax.numpy as jnp
from jax import lax
from jax.experimental import pallas as pl
from jax.experimental.pallas import tpu as pltpu
from jax.experimental.pallas import tpu_sc as plsc

_BETA = 0.25
_BB = 512


def _vq_body(e_ref, esq_ref, w2t_ref, wsq_ref, idx_ref, loss_ref):
    e = e_ref[...]
    mm2_full = jnp.dot(e, w2t_ref[...],
                       preferred_element_type=jnp.float32)
    k = mm2_full.shape[1]
    ns = k // 128
    bb = 128
    wsq = wsq_ref[...]
    lane = lax.broadcasted_iota(jnp.int32, (bb, 128), 1)

    def half_argmin(mm2, esq_b, s_lo, s_hi):
        nacc = 1
        avs = [jnp.full((bb, 128), jnp.inf, jnp.float32) for _ in range(nacc)]
        ais = [jnp.zeros((bb, 128), jnp.int32) for _ in range(nacc)]
        for s in range(s_lo, s_hi):
            a = s % nacc
            d2s = (esq_b - mm2[:, s * 128:(s + 1) * 128]) + jnp.broadcast_to(
                wsq[:, s * 128:(s + 1) * 128], (bb, 128))
            upd = d2s < avs[a]
            avs[a] = jnp.where(upd, d2s, avs[a])
            ais[a] = jnp.where(upd, s, ais[a])
        av, ai = avs[0], ais[0]
        for a in range(1, nacc):
            take = (avs[a] < av) | ((avs[a] == av) & (ais[a] < ai))
            av = jnp.where(take, avs[a], av)
            ai = jnp.where(take, ais[a], ai)
        v = jnp.min(av, axis=1, keepdims=True)
        kfull = ai * 128 + lane
        i = jnp.min(jnp.where(av == v, kfull, k), axis=1)
        return v, i

    acc = jnp.zeros((1, 1), jnp.float32)
    for rb in range(_BB // bb):
        rows = slice(rb * bb, (rb + 1) * bb)
        mm2 = mm2_full[rows, :]
        esq_b = jnp.broadcast_to(esq_ref[rows, :], (bb, 128))
        vl, il = half_argmin(mm2, esq_b, 0, ns // 2)
        vr, ir = half_argmin(mm2, esq_b, ns // 2, ns)
        vl_bf = vl.astype(jnp.bfloat16).astype(jnp.float32)
        pickr = vr < vl_bf
        idx_ref[0, 0, rows] = jnp.where(pickr[:, 0], ir, il)
        dsel = jnp.where(pickr[:, 0], vr[:, 0], vl[:, 0])
        acc = acc + jnp.sum(dsel).reshape(1, 1)
    @pl.when(pl.program_id(0) == 0)
    def _():
        loss_ref[...] = jnp.zeros((1, 1), jnp.float32)
    loss_ref[...] += acc


def _argmin_losses(encoding, embedding_weight):
    b, d = encoding.shape
    k = embedding_weight.shape[0]
    enc_sq = jnp.sum(encoding * encoding, axis=1, keepdims=True)
    w_sq = jnp.sum(embedding_weight * embedding_weight, axis=1)
    w2t = (2.0 * embedding_weight).T
    g = b // _BB
    idx3, loss = pl.pallas_call(
        _vq_body,
        grid=(g,),
        in_specs=[
            pl.BlockSpec((_BB, d), lambda i: (i, 0)),
            pl.BlockSpec((_BB, 1), lambda i: (i, 0)),
            pl.BlockSpec((d, k), lambda i: (0, 0)),
            pl.BlockSpec((1, k), lambda i: (0, 0)),
        ],
        out_specs=[
            pl.BlockSpec((1, 1, _BB), lambda i: (i, 0, 0)),
            pl.BlockSpec((1, 1), lambda i: (0, 0)),
        ],
        out_shape=[
            jax.ShapeDtypeStruct((g, 1, _BB), jnp.int32),
            jax.ShapeDtypeStruct((1, 1), jnp.float32),
        ],
        compiler_params=pltpu.CompilerParams(
            dimension_semantics=("arbitrary",),
        ),
    )(encoding, enc_sq, w2t, w_sq.reshape(1, k))
    return idx3.reshape(b), loss[0, 0]


def _make_sc_gather(b, k, d):
    info = plsc.get_sparse_core_info()
    nw = info.num_cores * info.num_subcores
    b_per_w = b // nw
    nc = b_per_w // 128
    nbuf = 6
    mesh = plsc.VectorSubcoreMesh(core_axis_name="c", subcore_axis_name="s")

    @functools.partial(
        pl.kernel, mesh=mesh,
        out_type=jax.ShapeDtypeStruct((b, 128), jnp.float32),
        scratch_types=(
            [pltpu.VMEM((b_per_w,), jnp.int32)]
            + [pltpu.VMEM((128, 128), jnp.float32)] * nbuf
            + [pltpu.SemaphoreType.DMA] * (2 * nbuf)
        ),
    )
    def gather_kernel(table_hbm, idx_hbm, out_hbm, idx_v, *rest):
        bufs = rest[:nbuf]
        gsem = rest[nbuf:2 * nbuf]
        wsem = rest[2 * nbuf:3 * nbuf]
        wid = lax.axis_index("s") * info.num_cores + lax.axis_index("c")
        base = wid * b_per_w
        pltpu.sync_copy(idx_hbm.at[pl.ds(base, b_per_w)], idx_v)

        hg = [None] * nbuf
        hw = [None] * nbuf
        for j in range(min(nbuf, nc)):
            hg[j] = pltpu.async_copy(
                table_hbm.at[idx_v.at[pl.ds(j * 128, 128)]], bufs[j], gsem[j])
        for j in range(nc):
            q = j % nbuf
            hg[q].wait()
            hw[q] = pltpu.async_copy(
                bufs[q], out_hbm.at[pl.ds(base + j * 128, 128)], wsem[q])
            nxt = j + nbuf
            if nxt < nc:
                hw[q].wait()
                hg[q] = pltpu.async_copy(
                    table_hbm.at[idx_v.at[pl.ds(nxt * 128, 128)]],
                    bufs[q], gsem[q])
        for j in range(max(0, nc - nbuf), nc):
            hw[j % nbuf].wait()

    return gather_kernel


def kernel(encoding, embedding_weight):
    b, d = encoding.shape
    k = embedding_weight.shape[0]
    quantized_index, loss_sum = _argmin_losses(encoding, embedding_weight)
    table_p = jnp.pad(embedding_weight, ((0, 0), (0, 128 - d)))
    quantized_st = _make_sc_gather(b, k, d)(table_p, quantized_index)[:, :d]
    s = loss_sum / (b * d)
    commitment_loss = s
    embedding_loss = s
    vq_loss = commitment_loss * _BETA + embedding_loss
    return (quantized_index, quantized_st, vq_loss, embedding_loss, commitment_loss)
